# Initial kernel scaffold; baseline (speedup 1.0000x reference)
import jax
import jax.numpy as jnp
from jax.experimental import pallas as pl


def kernel(idx, vocab_table, pos_table):
    raise NotImplementedError("write your pallas kernel here")


# speedup vs baseline: 2.1252x; 2.1252x over previous
import functools

import jax
import jax.numpy as jnp
from jax import lax
from jax.experimental import pallas as pl
from jax.experimental.pallas import tpu as pltpu
from jax.experimental.pallas import tpu_sc as plsc


def _emb_kernel_body(B, T, D, NW, C, SG, idx_hbm, vocab_hbm, pos_hbm, out_hbm,
                     idx0, idx1, rows0, rows1, pos_v, sem0, sem1):
    K = C // SG
    R = C // T
    per_w = (B * T) // NW
    nchunks = per_w // C

    wid = lax.axis_index("s") * 2 + lax.axis_index("c")
    w_base = wid * per_w

    pltpu.sync_copy(pos_hbm.at[pl.ds(0, T)], pos_v)

    def load_and_fire(c, idxbuf, rowsbuf, sem):
        pltpu.sync_copy(idx_hbm.at[pl.ds(w_base + c * C, C)], idxbuf)
        for j in range(K):
            pltpu.async_copy(vocab_hbm.at[idxbuf.at[pl.ds(j * SG, SG)]],
                             rowsbuf.at[pl.ds(j * SG, SG)], sem)

    def wait_gathers(rowsbuf, sem):
        pltpu.make_async_copy(vocab_hbm.at[pl.ds(0, C)], rowsbuf, sem).wait()

    def add_pos(rowsbuf):
        def body(t, carry):
            for r in range(R):
                for j in range(D // 16):
                    sl = pl.ds(j * 16, 16)
                    rowsbuf[r * T + t, sl] = rowsbuf[r * T + t, sl] + pos_v[t, sl]
            return carry
        lax.fori_loop(0, T, body, 0, unroll=2)

    def scatter(c, rowsbuf):
        pltpu.sync_copy(rowsbuf, out_hbm.at[pl.ds(w_base + c * C, C)])

    load_and_fire(0, idx0, rows0, sem0)

    def outer(g, carry):
        c0 = 2 * g
        c1 = 2 * g + 1
        load_and_fire(c1, idx1, rows1, sem1)
        wait_gathers(rows0, sem0)
        add_pos(rows0)
        scatter(c0, rows0)

        @pl.when(g < nchunks // 2 - 1)
        def _():
            load_and_fire(c0 + 2, idx0, rows0, sem0)

        wait_gathers(rows1, sem1)
        add_pos(rows1)
        scatter(c1, rows1)
        return carry

    lax.fori_loop(0, nchunks // 2, outer, 0)


def kernel(idx, vocab_table, pos_table):
    B, T = idx.shape
    V, D = vocab_table.shape
    NW = 32
    C = 2 * T
    SG = 80

    idx_flat = idx.reshape(-1).astype(jnp.int32)

    mesh = plsc.VectorSubcoreMesh(core_axis_name="c", subcore_axis_name="s",
                                  num_cores=2, num_subcores=16)
    run = functools.partial(
        pl.kernel,
        out_type=jax.ShapeDtypeStruct((B * T, D), jnp.float32),
        mesh=mesh,
        scratch_types=[
            pltpu.VMEM((C,), jnp.int32),
            pltpu.VMEM((C,), jnp.int32),
            pltpu.VMEM((C, D), jnp.float32),
            pltpu.VMEM((C, D), jnp.float32),
            pltpu.VMEM((T, D), jnp.float32),
            pltpu.SemaphoreType.DMA,
            pltpu.SemaphoreType.DMA,
        ],
        compiler_params=pltpu.CompilerParams(use_tc_tiling_on_sc=False),
    )(functools.partial(_emb_kernel_body, B, T, D, NW, C, SG))

    out = run(idx_flat, vocab_table, pos_table)
    return out.reshape(B, T, D)


# speedup vs baseline: 2.2585x; 1.0627x over previous
import functools

import jax
import jax.numpy as jnp
from jax import lax
from jax.experimental import pallas as pl
from jax.experimental.pallas import tpu as pltpu
from jax.experimental.pallas import tpu_sc as plsc


def _emb_kernel_body(B, T, D, NW, C, SG, idx_hbm, vocab_hbm, pos_hbm, out_hbm,
                     idx_v, rows0, rows1, rows2, pos_v,
                     sg0, sg1, sg2, ss0, ss1, ss2):
    K = C // SG
    R = C // T
    per_w = (B * T) // NW
    nchunks = per_w // C

    rows = (rows0, rows1, rows2)
    gsems = (sg0, sg1, sg2)
    ssems = (ss0, ss1, ss2)

    wid = lax.axis_index("s") * 2 + lax.axis_index("c")
    w_base = wid * per_w

    pltpu.sync_copy(pos_hbm.at[pl.ds(0, T)], pos_v)
    pltpu.sync_copy(idx_hbm.at[pl.ds(w_base, per_w)], idx_v)

    def fire_gather(c, b):
        for j in range(K):
            pltpu.async_copy(vocab_hbm.at[idx_v.at[pl.ds(c * C + j * SG, SG)]],
                             rows[b].at[pl.ds(j * SG, SG)], gsems[b])

    def wait_gather(b):
        pltpu.make_async_copy(vocab_hbm.at[pl.ds(0, C)], rows[b], gsems[b]).wait()

    def add_pos(b):
        rbuf = rows[b]
        def body(t, carry):
            for r in range(R):
                for j in range(D // 16):
                    sl = pl.ds(j * 16, 16)
                    rbuf[r * T + t, sl] = rbuf[r * T + t, sl] + pos_v[t, sl]
            return carry
        lax.fori_loop(0, T, body, 0, unroll=2)

    def fire_scatter(c, b):
        pltpu.async_copy(rows[b], out_hbm.at[pl.ds(w_base + c * C, C)], ssems[b])

    def wait_scatter(b):
        pltpu.make_async_copy(rows[b], out_hbm.at[pl.ds(0, C)], ssems[b]).wait()

    NG = nchunks // 3
    last_g = NG - 1

    fire_gather(0, 0)
    fire_gather(1, 1)

    def outer(g, carry):
        c0 = 3 * g
        wait_gather(0)
        add_pos(0)
        fire_scatter(c0, 0)
        @pl.when(g > 0)
        def _():
            wait_scatter(2)
        fire_gather(c0 + 2, 2)
        wait_gather(1)
        add_pos(1)
        fire_scatter(c0 + 1, 1)
        wait_scatter(0)
        fire_gather(c0 + 3, 0)
        wait_gather(2)
        add_pos(2)
        fire_scatter(c0 + 2, 2)
        wait_scatter(1)
        @pl.when(g < last_g)
        def _():
            fire_gather(c0 + 4, 1)
        return carry

    lax.fori_loop(0, NG, outer, 0)

    wait_gather(0)
    add_pos(0)
    fire_scatter(nchunks - 1, 0)
    wait_scatter(2)
    wait_scatter(0)


def kernel(idx, vocab_table, pos_table):
    B, T = idx.shape
    V, D = vocab_table.shape
    NW = 32
    C = 2 * T
    SG = 80

    idx_flat = idx.reshape(-1).astype(jnp.int32)

    mesh = plsc.VectorSubcoreMesh(core_axis_name="c", subcore_axis_name="s",
                                  num_cores=2, num_subcores=16)
    run = functools.partial(
        pl.kernel,
        out_type=jax.ShapeDtypeStruct((B * T, D), jnp.float32),
        mesh=mesh,
        scratch_types=[
            pltpu.VMEM(((B * T) // NW,), jnp.int32),
            pltpu.VMEM((C, D), jnp.float32),
            pltpu.VMEM((C, D), jnp.float32),
            pltpu.VMEM((C, D), jnp.float32),
            pltpu.VMEM((T, D), jnp.float32),
            pltpu.SemaphoreType.DMA,
            pltpu.SemaphoreType.DMA,
            pltpu.SemaphoreType.DMA,
            pltpu.SemaphoreType.DMA,
            pltpu.SemaphoreType.DMA,
            pltpu.SemaphoreType.DMA,
        ],
        compiler_params=pltpu.CompilerParams(use_tc_tiling_on_sc=False),
    )(functools.partial(_emb_kernel_body, B, T, D, NW, C, SG))

    out = run(idx_flat, vocab_table, pos_table)
    return out.reshape(B, T, D)


# speedup vs baseline: 2.2601x; 1.0007x over previous
import functools

import jax
import jax.numpy as jnp
from jax import lax
from jax.experimental import pallas as pl
from jax.experimental.pallas import tpu as pltpu
from jax.experimental.pallas import tpu_sc as plsc

_NBUF = 5


def _emb_kernel_body(B, T, D, NW, C, SG, idx_hbm, vocab_hbm, pos_hbm, out_hbm,
                     idx_v, pos_v, *bufs_and_sems):
    rows = bufs_and_sems[:_NBUF]
    gsems = bufs_and_sems[_NBUF:2 * _NBUF]
    ssems = bufs_and_sems[2 * _NBUF:3 * _NBUF]
    K = C // SG
    per_w = (B * T) // NW
    nchunks = per_w // C
    NG = nchunks // _NBUF

    wid = lax.axis_index("s") * 2 + lax.axis_index("c")
    w_base = wid * per_w

    pltpu.sync_copy(pos_hbm.at[pl.ds(0, T)], pos_v)
    pltpu.sync_copy(idx_hbm.at[pl.ds(w_base, per_w)], idx_v)

    def fire_gather(c, b):
        for j in range(K):
            pltpu.async_copy(vocab_hbm.at[idx_v.at[pl.ds(c * C + j * SG, SG)]],
                             rows[b].at[pl.ds(j * SG, SG)], gsems[b])

    def wait_gather(b):
        pltpu.make_async_copy(vocab_hbm.at[pl.ds(0, C)], rows[b], gsems[b]).wait()

    def add_pos(c, b):
        rbuf = rows[b]
        t0 = lax.rem(c * C, T)
        def body(l, carry):
            t = lax.rem(t0 + l, T)
            for j in range(D // 16):
                sl = pl.ds(j * 16, 16)
                rbuf[l, sl] = rbuf[l, sl] + pos_v[t, sl]
            return carry
        lax.fori_loop(0, C, body, 0, unroll=4)

    def fire_scatter(c, b):
        pltpu.async_copy(rows[b], out_hbm.at[pl.ds(w_base + c * C, C)], ssems[b])

    def wait_scatter(b):
        pltpu.make_async_copy(rows[b], out_hbm.at[pl.ds(0, C)], ssems[b]).wait()

    for b in range(3):
        fire_gather(b, b)

    def outer(g, carry):
        c0 = _NBUF * g
        for i in range(_NBUF):
            b3 = (i + 3) % _NBUF
            wait_gather(i)
            add_pos(c0 + i, i)
            fire_scatter(c0 + i, i)
            if i < 2:
                @pl.when(g > 0)
                def _(b3=b3):
                    wait_scatter(b3)
            else:
                wait_scatter(b3)
            if i < 2:
                fire_gather(c0 + i + 3, b3)
            else:
                @pl.when(g < NG - 1)
                def _(c=c0 + i + 3, b3=b3):
                    fire_gather(c, b3)
        return carry

    lax.fori_loop(0, NG, outer, 0)

    wait_scatter((nchunks - 2) % _NBUF)
    wait_scatter((nchunks - 1) % _NBUF)


def kernel(idx, vocab_table, pos_table):
    B, T = idx.shape
    V, D = vocab_table.shape
    NW = 32
    C = 256
    SG = 128

    idx_flat = idx.reshape(-1).astype(jnp.int32)

    mesh = plsc.VectorSubcoreMesh(core_axis_name="c", subcore_axis_name="s",
                                  num_cores=2, num_subcores=16)
    run = functools.partial(
        pl.kernel,
        out_type=jax.ShapeDtypeStruct((B * T, D), jnp.float32),
        mesh=mesh,
        scratch_types=[
            pltpu.VMEM(((B * T) // NW,), jnp.int32),
            pltpu.VMEM((T, D), jnp.float32),
            *[pltpu.VMEM((C, D), jnp.float32) for _ in range(_NBUF)],
            *[pltpu.SemaphoreType.DMA for _ in range(2 * _NBUF)],
        ],
        compiler_params=pltpu.CompilerParams(use_tc_tiling_on_sc=False),
    )(functools.partial(_emb_kernel_body, B, T, D, NW, C, SG))

    out = run(idx_flat, vocab_table, pos_table)
    return out.reshape(B, T, D)


# speedup vs baseline: 2.8697x; 1.2697x over previous
import functools

import jax
import jax.numpy as jnp
from jax import lax
from jax.experimental import pallas as pl
from jax.experimental.pallas import tpu as pltpu
from jax.experimental.pallas import tpu_sc as plsc

_NBUF = 5


def _emb_kernel_body(B, T, D, NW, C, SG, idx_hbm, vocab_hbm, pos_hbm, out_hbm,
                     idx_v, pos_v, pos_sh, *bufs_and_sems):
    rows = bufs_and_sems[:_NBUF]
    gsems = bufs_and_sems[_NBUF:2 * _NBUF]
    ssems = bufs_and_sems[2 * _NBUF:3 * _NBUF]
    psems = bufs_and_sems[3 * _NBUF:4 * _NBUF]
    K = C // SG
    per_w = (B * T) // NW
    nchunks = per_w // C
    NG = nchunks // _NBUF

    wid = lax.axis_index("s") * 2 + lax.axis_index("c")
    w_base = wid * per_w

    pltpu.sync_copy(idx_hbm.at[pl.ds(w_base, per_w)], idx_v)

    @pl.when(lax.axis_index("s") == 0)
    def _():
        pltpu.sync_copy(pos_hbm.at[pl.ds(0, T)], pos_v)
        pltpu.sync_copy(pos_v, pos_sh)
    plsc.subcore_barrier()

    def fire_prefill(b):
        pltpu.async_copy(pos_sh, rows[b], psems[b])

    def wait_prefill(b):
        pltpu.make_async_copy(pos_sh, rows[b], psems[b]).wait()

    def fire_gather(c, b):
        for j in range(K):
            pltpu.async_copy(vocab_hbm.at[idx_v.at[pl.ds(c * C + j * SG, SG)]],
                             rows[b].at[pl.ds(j * SG, SG)], gsems[b],
                             add=True)

    def wait_gather(b):
        pltpu.make_async_copy(vocab_hbm.at[pl.ds(0, C)], rows[b], gsems[b]).wait()

    def fire_scatter(c, b):
        pltpu.async_copy(rows[b], out_hbm.at[pl.ds(w_base + c * C, C)], ssems[b])

    def wait_scatter(b):
        pltpu.make_async_copy(rows[b], out_hbm.at[pl.ds(0, C)], ssems[b]).wait()

    for b in range(3):
        fire_prefill(b)
        wait_prefill(b)
        fire_gather(b, b)
    fire_prefill(3)

    def outer(g, carry):
        c0 = _NBUF * g
        for i in range(_NBUF):
            c = c0 + i
            b = i
            b3 = (i + 3) % _NBUF
            b4 = (i + 4) % _NBUF
            if i == 0:
                @pl.when(g > 0)
                def _(b4=b4):
                    wait_scatter(b4)
            else:
                wait_scatter(b4)
            @pl.when(_NBUF * g + i + 4 < nchunks)
            def _(b4=b4):
                fire_prefill(b4)
            wait_prefill(b3)
            fire_gather(c + 3, b3)
            wait_gather(b)
            fire_scatter(c, b)
        return carry

    lax.fori_loop(0, nchunks // _NBUF, outer, 0)

    for c in range(nchunks - nchunks % _NBUF, nchunks):
        b = c % _NBUF
        wait_gather(b)
        fire_scatter(c, b)

    for c in range(nchunks - 4, nchunks):
        wait_scatter(c % _NBUF)


def kernel(idx, vocab_table, pos_table):
    B, T = idx.shape
    V, D = vocab_table.shape
    NW = 32
    C = T
    SG = 40

    idx_flat = idx.reshape(-1).astype(jnp.int32)

    mesh = plsc.VectorSubcoreMesh(core_axis_name="c", subcore_axis_name="s",
                                  num_cores=2, num_subcores=16)
    run = functools.partial(
        pl.kernel,
        out_type=jax.ShapeDtypeStruct((B * T, D), jnp.float32),
        mesh=mesh,
        scratch_types=[
            pltpu.VMEM(((B * T) // NW,), jnp.int32),
            pltpu.VMEM((T, D), jnp.float32),
            pltpu.VMEM_SHARED((T, D), jnp.float32),
            *[pltpu.VMEM((C, D), jnp.float32) for _ in range(_NBUF)],
            *[pltpu.SemaphoreType.DMA for _ in range(3 * _NBUF)],
        ],
        compiler_params=pltpu.CompilerParams(use_tc_tiling_on_sc=False),
    )(functools.partial(_emb_kernel_body, B, T, D, NW, C, SG))

    out = run(idx_flat, vocab_table, pos_table)
    return out.reshape(B, T, D)
